# Initial kernel scaffold; baseline (speedup 1.0000x reference)
import jax
import jax.numpy as jnp
from jax.experimental import pallas as pl


def kernel(hidden_states, W_gate):
    raise NotImplementedError("write your pallas kernel here")


# speedup vs baseline: 5.1726x; 5.1726x over previous
import functools

import jax
import jax.numpy as jnp
from jax.experimental import pallas as pl
from jax.experimental.pallas import tpu as pltpu

E = 8
K = 2
H = 1024
T = 2048
CAP = 640
AUX_COEF = 0.01
Z_COEF = 0.001
BT = 256
G = T // BT


def _gate_kernel(x_ref, w_ref, dispatch_ref, combine_ref, idx_ref,
                 lb_ref, z_ref, counts_ref, psum_ref, zsum_ref):
    i = pl.program_id(0)

    @pl.when(i == 0)
    def _init():
        counts_ref[...] = jnp.zeros_like(counts_ref)
        psum_ref[...] = jnp.zeros_like(psum_ref)
        zsum_ref[...] = jnp.zeros_like(zsum_ref)

    x = x_ref[...]
    w = w_ref[...]
    logits = jax.lax.dot_general(
        x, w, (((1,), (1,)), ((), ())),
        preferred_element_type=jnp.float32)

    col = jax.lax.broadcasted_iota(jnp.int32, (BT, E), 1)
    m0 = jnp.max(logits, axis=1, keepdims=True)
    i0 = jnp.min(jnp.where(logits == m0, col, E), axis=1, keepdims=True)
    masked = jnp.where(col == i0, -jnp.inf, logits)
    m1 = jnp.max(masked, axis=1, keepdims=True)
    i1 = jnp.min(jnp.where(masked == m1, col, E), axis=1, keepdims=True)

    t = jnp.exp(m1 - m0)
    w0 = 1.0 / (1.0 + t)
    w1 = t / (1.0 + t)

    ex = jnp.exp(logits - m0)
    zdenom = jnp.sum(ex, axis=1, keepdims=True)
    probs = ex / zdenom
    psum_ref[...] += jnp.sum(probs, axis=0, keepdims=True)
    zsum_ref[...] += jnp.sum(m0 + jnp.log(zdenom)).reshape(1, 1)

    a = (col == i0).astype(jnp.float32) + (col == i1).astype(jnp.float32)

    r_i = jax.lax.broadcasted_iota(jnp.int32, (BT, BT), 0)
    c_i = jax.lax.broadcasted_iota(jnp.int32, (BT, BT), 1)
    tri = (r_i > c_i).astype(jnp.float32)
    c_local = jax.lax.dot_general(
        tri, a, (((1,), (0,)), ((), ())),
        preferred_element_type=jnp.float32)
    c_global = c_local + counts_ref[...]

    p0 = jnp.sum(jnp.where(col == i0, c_global, 0.0), axis=1, keepdims=True)
    p1 = jnp.sum(jnp.where(col == i1, c_global, 0.0), axis=1, keepdims=True)
    p0 = p0.astype(jnp.int32)
    p1 = p1.astype(jnp.int32)

    counts_ref[...] += jnp.sum(a, axis=0, keepdims=True)

    q0 = jnp.where(p0 < CAP, i0 * CAP + p0, -1)
    q1 = jnp.where(p1 < CAP, i1 * CAP + p1, -1)
    cq = jax.lax.broadcasted_iota(jnp.int32, (BT, E * CAP), 1)
    hit0 = cq == q0
    hit1 = cq == q1
    dispatch_ref[...] = hit0 | hit1
    combine_ref[...] = jnp.where(hit0, w0, 0.0) + jnp.where(hit1, w1, 0.0)

    idx_ref[...] = jnp.concatenate([i0, i1], axis=1)

    tpe = jnp.minimum(counts_ref[...], float(CAP))
    tpe = tpe / jnp.sum(tpe)
    mean_prob = psum_ref[...] / float(T)
    lb_ref[...] = (AUX_COEF * E * jnp.sum(mean_prob * tpe)).reshape(1, 1)
    z_ref[...] = (Z_COEF * zsum_ref[...] / float(T)).reshape(1, 1)


@jax.jit
def kernel(hidden_states, W_gate):
    x = hidden_states.reshape(T, H)
    dispatch, combine, idx, lb, z = pl.pallas_call(
        _gate_kernel,
        grid=(G,),
        in_specs=[
            pl.BlockSpec((BT, H), lambda i: (i, 0)),
            pl.BlockSpec((E, H), lambda i: (0, 0)),
        ],
        out_specs=[
            pl.BlockSpec((BT, E * CAP), lambda i: (i, 0)),
            pl.BlockSpec((BT, E * CAP), lambda i: (i, 0)),
            pl.BlockSpec((BT, K), lambda i: (i, 0)),
            pl.BlockSpec((1, 1), lambda i: (0, 0)),
            pl.BlockSpec((1, 1), lambda i: (0, 0)),
        ],
        out_shape=[
            jax.ShapeDtypeStruct((T, E * CAP), jnp.bool_),
            jax.ShapeDtypeStruct((T, E * CAP), jnp.float32),
            jax.ShapeDtypeStruct((T, K), jnp.int32),
            jax.ShapeDtypeStruct((1, 1), jnp.float32),
            jax.ShapeDtypeStruct((1, 1), jnp.float32),
        ],
        scratch_shapes=[
            pltpu.VMEM((1, E), jnp.float32),
            pltpu.VMEM((1, E), jnp.float32),
            pltpu.VMEM((1, 1), jnp.float32),
        ],
        compiler_params=pltpu.CompilerParams(
            dimension_semantics=("arbitrary",),
        ),
    )(x, W_gate)
    return (dispatch.reshape(T, E, CAP), combine.reshape(T, E, CAP),
            idx, lb.reshape(()), z.reshape(()))


# speedup vs baseline: 7.8731x; 1.5221x over previous
import functools

import jax
import jax.numpy as jnp
from jax.experimental import pallas as pl
from jax.experimental.pallas import tpu as pltpu

E = 8
K = 2
H = 1024
T = 2048
CAP = 640
AUX_COEF = 0.01
Z_COEF = 0.001
BT = 256
G = T // BT


def _gate_kernel(x_ref, w_ref, dispatch_ref, combine_ref, idx_ref,
                 lb_ref, z_ref, counts_ref, psum_ref, zsum_ref):
    i = pl.program_id(0)

    @pl.when(i == 0)
    def _init():
        counts_ref[...] = jnp.zeros_like(counts_ref)
        psum_ref[...] = jnp.zeros_like(psum_ref)
        zsum_ref[...] = jnp.zeros_like(zsum_ref)

    x = x_ref[...]
    w = w_ref[...]
    logits = jax.lax.dot_general(
        x, w, (((1,), (1,)), ((), ())),
        preferred_element_type=jnp.float32)

    col = jax.lax.broadcasted_iota(jnp.int32, (BT, E), 1)
    m0 = jnp.max(logits, axis=1, keepdims=True)
    i0 = jnp.min(jnp.where(logits == m0, col, E), axis=1, keepdims=True)
    masked = jnp.where(col == i0, -jnp.inf, logits)
    m1 = jnp.max(masked, axis=1, keepdims=True)
    i1 = jnp.min(jnp.where(masked == m1, col, E), axis=1, keepdims=True)

    t = jnp.exp(m1 - m0)
    w0 = 1.0 / (1.0 + t)
    w1 = t / (1.0 + t)

    ex = jnp.exp(logits - m0)
    zdenom = jnp.sum(ex, axis=1, keepdims=True)
    probs = ex / zdenom
    psum_ref[...] += jnp.sum(probs, axis=0, keepdims=True)
    zsum_ref[...] += jnp.sum(m0 + jnp.log(zdenom)).reshape(1, 1)

    a = (col == i0).astype(jnp.float32) + (col == i1).astype(jnp.float32)

    r_i = jax.lax.broadcasted_iota(jnp.int32, (BT, BT), 0)
    c_i = jax.lax.broadcasted_iota(jnp.int32, (BT, BT), 1)
    tri = (r_i > c_i).astype(jnp.float32)
    c_local = jax.lax.dot_general(
        tri, a, (((1,), (0,)), ((), ())),
        preferred_element_type=jnp.float32)
    c_global = c_local + counts_ref[...]

    p0 = jnp.sum(jnp.where(col == i0, c_global, 0.0), axis=1, keepdims=True)
    p1 = jnp.sum(jnp.where(col == i1, c_global, 0.0), axis=1, keepdims=True)
    p0 = p0.astype(jnp.int32)
    p1 = p1.astype(jnp.int32)

    counts_ref[...] += jnp.sum(a, axis=0, keepdims=True)

    q0 = jnp.where(p0 < CAP, i0 * CAP + p0, -1)
    q1 = jnp.where(p1 < CAP, i1 * CAP + p1, -1)
    cq = jax.lax.broadcasted_iota(jnp.int32, (BT, E * CAP), 1)
    hit0 = cq == q0
    hit1 = cq == q1
    d2 = hit0 | hit1
    c2 = jnp.where(hit0, w0, 0.0) + jnp.where(hit1, w1, 0.0)
    for e in range(E):
        dispatch_ref[:, e, :] = d2[:, e * CAP:(e + 1) * CAP]
        combine_ref[:, e, :] = c2[:, e * CAP:(e + 1) * CAP]

    idx_ref[...] = jnp.concatenate([i0, i1], axis=1)

    tpe = jnp.minimum(counts_ref[...], float(CAP))
    tpe = tpe / jnp.sum(tpe)
    mean_prob = psum_ref[...] / float(T)
    lb_ref[...] = (AUX_COEF * E * jnp.sum(mean_prob * tpe)).reshape(1, 1)
    z_ref[...] = (Z_COEF * zsum_ref[...] / float(T)).reshape(1, 1)


@jax.jit
def kernel(hidden_states, W_gate):
    x = hidden_states.reshape(T, H)
    dispatch, combine, idx, lb, z = pl.pallas_call(
        _gate_kernel,
        grid=(G,),
        in_specs=[
            pl.BlockSpec((BT, H), lambda i: (i, 0)),
            pl.BlockSpec((E, H), lambda i: (0, 0)),
        ],
        out_specs=[
            pl.BlockSpec((BT, E, CAP), lambda i: (i, 0, 0)),
            pl.BlockSpec((BT, E, CAP), lambda i: (i, 0, 0)),
            pl.BlockSpec((BT, K), lambda i: (i, 0)),
            pl.BlockSpec((1, 1), lambda i: (0, 0)),
            pl.BlockSpec((1, 1), lambda i: (0, 0)),
        ],
        out_shape=[
            jax.ShapeDtypeStruct((T, E, CAP), jnp.bool_),
            jax.ShapeDtypeStruct((T, E, CAP), jnp.float32),
            jax.ShapeDtypeStruct((T, K), jnp.int32),
            jax.ShapeDtypeStruct((1, 1), jnp.float32),
            jax.ShapeDtypeStruct((1, 1), jnp.float32),
        ],
        scratch_shapes=[
            pltpu.VMEM((1, E), jnp.float32),
            pltpu.VMEM((1, E), jnp.float32),
            pltpu.VMEM((1, 1), jnp.float32),
        ],
        compiler_params=pltpu.CompilerParams(
            dimension_semantics=("arbitrary",),
        ),
    )(x, W_gate)
    return dispatch, combine, idx, lb.reshape(()), z.reshape(())
